# Initial kernel scaffold; baseline (speedup 1.0000x reference)
import functools

import jax
import jax.numpy as jnp
from jax import lax
from jax.experimental import pallas as pl
from jax.experimental.pallas import tpu as pltpu
from jax.experimental.pallas import tpu_sc as plsc

N_NODES = 10000
N_EDGES = 320000
HIDDEN = 128
EDGE_DIM = 16

NC = 2
NS = 16
NW = NC * NS
PER_W = N_EDGES // NW
CHUNK = 80
NCHUNK = PER_W // CHUNK
ROWS_PER_TILE = N_NODES // NS
ZROWS = 125


def _edge_mlp_body(a_ref, w_ref, b_ref, o_ref):
    o_ref[...] = (
        jnp.dot(a_ref[...], w_ref[...], preferred_element_type=jnp.float32)
        + b_ref[...]
    )


def _edge_mlp(edge_attr, W_e, b_e):
    be = 4000
    grid = N_EDGES // be
    return pl.pallas_call(
        _edge_mlp_body,
        grid=(grid,),
        in_specs=[
            pl.BlockSpec((be, EDGE_DIM), lambda i: (i, 0)),
            pl.BlockSpec((EDGE_DIM, HIDDEN), lambda i: (0, 0)),
            pl.BlockSpec((1, HIDDEN), lambda i: (0, 0)),
        ],
        out_specs=pl.BlockSpec((be, HIDDEN), lambda i: (i, 0)),
        out_shape=jax.ShapeDtypeStruct((N_EDGES, HIDDEN), jnp.float32),
    )(edge_attr, W_e, b_e.reshape(1, HIDDEN))


def _sc_body(x_hbm, src_hbm, dst_hbm, e_hbm, out_hbm,
             srcb, dstb, ebuf, xbuf, zbuf, agg, sem):
    cid = lax.axis_index("c")
    sid = lax.axis_index("s")
    wid = cid * NS + sid
    base = wid * PER_W

    zero16 = jnp.zeros((16,), jnp.float32)

    def zfill(i, _):
        for k in range(HIDDEN // 16):
            zbuf[i, pl.ds(k * 16, 16)] = zero16
        return 0

    lax.fori_loop(0, ZROWS, zfill, 0)
    for t in range(ROWS_PER_TILE // ZROWS):
        pltpu.sync_copy(
            zbuf, agg.at[pl.ds(sid * ROWS_PER_TILE + t * ZROWS, ZROWS)]
        )
    plsc.subcore_barrier()

    def chunk_body(c, _):
        off = base + c * CHUNK
        pltpu.sync_copy(src_hbm.at[pl.ds(off, CHUNK)], srcb)
        pltpu.sync_copy(dst_hbm.at[pl.ds(off, CHUNK)], dstb)
        pltpu.sync_copy(e_hbm.at[pl.ds(off, CHUNK)], ebuf)
        pltpu.async_copy(x_hbm.at[srcb], xbuf, sem).wait()

        def erow(i, _):
            for k in range(HIDDEN // 16):
                sl = pl.ds(k * 16, 16)
                ebuf[i, sl] = jnp.maximum(ebuf[i, sl] + xbuf[i, sl], 0.0)
            return 0

        lax.fori_loop(0, CHUNK, erow, 0)
        pltpu.sync_copy(ebuf, agg.at[dstb], add=True)
        return 0

    lax.fori_loop(0, NCHUNK, chunk_body, 0)
    plsc.subcore_barrier()

    rb = pl.ds(sid * ROWS_PER_TILE, ROWS_PER_TILE)
    pltpu.sync_copy(agg.at[rb], out_hbm.at[cid].at[rb])


def _sc_aggregate(x, src, dst, e):
    mesh = plsc.VectorSubcoreMesh(core_axis_name="c", subcore_axis_name="s")
    k = pl.kernel(
        _sc_body,
        out_type=jax.ShapeDtypeStruct((NC, N_NODES, HIDDEN), jnp.float32),
        mesh=mesh,
        scratch_types=[
            pltpu.VMEM((CHUNK,), jnp.int32),
            pltpu.VMEM((CHUNK,), jnp.int32),
            pltpu.VMEM((CHUNK, HIDDEN), jnp.float32),
            pltpu.VMEM((CHUNK, HIDDEN), jnp.float32),
            pltpu.VMEM((ZROWS, HIDDEN), jnp.float32),
            pltpu.VMEM_SHARED((N_NODES, HIDDEN), jnp.float32),
            pltpu.SemaphoreType.DMA,
        ],
    )
    return k(x, src, dst, e)


def _node_mlp_body(x_ref, a_ref, w1_ref, b1_ref, w2_ref, b2_ref,
                   g_ref, bt_ref, o_ref):
    x = x_ref[...]
    h = x + a_ref[0] + a_ref[1]
    h = jnp.maximum(
        jnp.dot(h, w1_ref[...], preferred_element_type=jnp.float32)
        + b1_ref[...], 0.0)
    h = (jnp.dot(h, w2_ref[...], preferred_element_type=jnp.float32)
         + b2_ref[...])
    mean = jnp.mean(h, axis=0, keepdims=True)
    var = jnp.mean((h - mean) ** 2, axis=0, keepdims=True)
    h = (h - mean) * lax.rsqrt(var + 1e-5) * g_ref[...] + bt_ref[...]
    o_ref[...] = jnp.maximum(h, 0.0) + x


def _node_mlp(x, aggs, W1, b1, W2, b2, gamma, beta):
    return pl.pallas_call(
        _node_mlp_body,
        in_specs=[
            pl.BlockSpec((N_NODES, HIDDEN), lambda: (0, 0)),
            pl.BlockSpec((NC, N_NODES, HIDDEN), lambda: (0, 0, 0)),
            pl.BlockSpec((HIDDEN, HIDDEN), lambda: (0, 0)),
            pl.BlockSpec((1, HIDDEN), lambda: (0, 0)),
            pl.BlockSpec((HIDDEN, HIDDEN), lambda: (0, 0)),
            pl.BlockSpec((1, HIDDEN), lambda: (0, 0)),
            pl.BlockSpec((1, HIDDEN), lambda: (0, 0)),
            pl.BlockSpec((1, HIDDEN), lambda: (0, 0)),
        ],
        out_specs=pl.BlockSpec((N_NODES, HIDDEN), lambda: (0, 0)),
        out_shape=jax.ShapeDtypeStruct((N_NODES, HIDDEN), jnp.float32),
    )(x, aggs, W1, b1.reshape(1, HIDDEN), W2, b2.reshape(1, HIDDEN),
      gamma.reshape(1, HIDDEN), beta.reshape(1, HIDDEN))


def kernel(x, edge_index, edge_attr, W_e, b_e, W1, b1, W2, b2, gamma, beta):
    src = edge_index[0].astype(jnp.int32)
    dst = edge_index[1].astype(jnp.int32)
    e = _edge_mlp(edge_attr, W_e, b_e)
    aggs = _sc_aggregate(x, src, dst, e)
    return _node_mlp(x, aggs, W1, b1, W2, b2, gamma, beta)


# speedup vs baseline: 2.4152x; 2.4152x over previous
import functools

import jax
import jax.numpy as jnp
from jax import lax
from jax.experimental import pallas as pl
from jax.experimental.pallas import tpu as pltpu
from jax.experimental.pallas import tpu_sc as plsc

N_NODES = 10000
N_EDGES = 320000
HIDDEN = 128
EDGE_DIM = 16

NC = 2
NS = 16
NW = NC * NS
PER_W = N_EDGES // NW
CHUNK = 80
NCHUNK = PER_W // CHUNK
AGG_ROWS = 10240
ROWS_PER_TILE = AGG_ROWS // NS
ZROWS = 128


def _edge_mlp_body(a_ref, w_ref, b_ref, o_ref):
    o_ref[...] = (
        jnp.dot(a_ref[...], w_ref[...], preferred_element_type=jnp.float32)
        + b_ref[...]
    )


def _edge_mlp(edge_attr, W_e, b_e):
    be = 4000
    grid = N_EDGES // be
    return pl.pallas_call(
        _edge_mlp_body,
        grid=(grid,),
        in_specs=[
            pl.BlockSpec((be, EDGE_DIM), lambda i: (i, 0)),
            pl.BlockSpec((EDGE_DIM, HIDDEN), lambda i: (0, 0)),
            pl.BlockSpec((1, HIDDEN), lambda i: (0, 0)),
        ],
        out_specs=pl.BlockSpec((be, HIDDEN), lambda i: (i, 0)),
        out_shape=jax.ShapeDtypeStruct((N_EDGES, HIDDEN), jnp.float32),
    )(edge_attr, W_e, b_e.reshape(1, HIDDEN))


def _sc_body(x_hbm, src_hbm, dst_hbm, e_hbm, out_hbm,
             srcb, dstb, ebuf, xbuf, zbuf, agg, sem):
    cid = lax.axis_index("c")
    sid = lax.axis_index("s")
    wid = cid * NS + sid
    base = wid * PER_W

    zero16 = jnp.zeros((16,), jnp.float32)

    def zfill(i, _):
        for k in range(HIDDEN // 16):
            zbuf[i, pl.ds(k * 16, 16)] = zero16
        return 0

    lax.fori_loop(0, ZROWS, zfill, 0)
    for t in range(ROWS_PER_TILE // ZROWS):
        pltpu.sync_copy(
            zbuf, agg.at[pl.ds(sid * ROWS_PER_TILE + t * ZROWS, ZROWS)]
        )
    plsc.subcore_barrier()

    def chunk_body(c, _):
        off = base + c * CHUNK
        pltpu.sync_copy(src_hbm.at[pl.ds(off, CHUNK)], srcb)
        pltpu.sync_copy(dst_hbm.at[pl.ds(off, CHUNK)], dstb)
        pltpu.sync_copy(e_hbm.at[pl.ds(off, CHUNK)], ebuf)
        pltpu.async_copy(x_hbm.at[srcb], xbuf, sem).wait()

        def erow(i, _):
            for k in range(HIDDEN // 16):
                sl = pl.ds(k * 16, 16)
                ebuf[i, sl] = jnp.maximum(ebuf[i, sl] + xbuf[i, sl], 0.0)
            return 0

        lax.fori_loop(0, CHUNK, erow, 0)
        pltpu.sync_copy(ebuf, agg.at[dstb], add=True)
        return 0

    lax.fori_loop(0, NCHUNK, chunk_body, 0)
    plsc.subcore_barrier()

    rb = pl.ds(sid * ROWS_PER_TILE, ROWS_PER_TILE)
    pltpu.sync_copy(agg.at[rb], out_hbm.at[cid].at[rb])


def _sc_aggregate(x, src, dst, e):
    mesh = plsc.VectorSubcoreMesh(core_axis_name="c", subcore_axis_name="s")
    k = pl.kernel(
        _sc_body,
        out_type=jax.ShapeDtypeStruct((NC, AGG_ROWS, HIDDEN), jnp.float32),
        mesh=mesh,
        scratch_types=[
            pltpu.VMEM((CHUNK,), jnp.int32),
            pltpu.VMEM((CHUNK,), jnp.int32),
            pltpu.VMEM((CHUNK, HIDDEN), jnp.float32),
            pltpu.VMEM((CHUNK, HIDDEN), jnp.float32),
            pltpu.VMEM((ZROWS, HIDDEN), jnp.float32),
            pltpu.VMEM_SHARED((AGG_ROWS, HIDDEN), jnp.float32),
            pltpu.SemaphoreType.DMA,
        ],
    )
    return k(x, src, dst, e)


def _node_mlp_body(x_ref, a_ref, w1_ref, b1_ref, w2_ref, b2_ref,
                   g_ref, bt_ref, o_ref):
    x = x_ref[...]
    h = x + a_ref[0] + a_ref[1]
    h = jnp.maximum(
        jnp.dot(h, w1_ref[...], preferred_element_type=jnp.float32)
        + b1_ref[...], 0.0)
    h = (jnp.dot(h, w2_ref[...], preferred_element_type=jnp.float32)
         + b2_ref[...])
    mean = jnp.mean(h, axis=0, keepdims=True)
    var = jnp.mean((h - mean) ** 2, axis=0, keepdims=True)
    h = (h - mean) * lax.rsqrt(var + 1e-5) * g_ref[...] + bt_ref[...]
    o_ref[...] = jnp.maximum(h, 0.0) + x


def _node_mlp(x, aggs, W1, b1, W2, b2, gamma, beta):
    return pl.pallas_call(
        _node_mlp_body,
        in_specs=[
            pl.BlockSpec((N_NODES, HIDDEN), lambda: (0, 0)),
            pl.BlockSpec((NC, N_NODES, HIDDEN), lambda: (0, 0, 0)),
            pl.BlockSpec((HIDDEN, HIDDEN), lambda: (0, 0)),
            pl.BlockSpec((1, HIDDEN), lambda: (0, 0)),
            pl.BlockSpec((HIDDEN, HIDDEN), lambda: (0, 0)),
            pl.BlockSpec((1, HIDDEN), lambda: (0, 0)),
            pl.BlockSpec((1, HIDDEN), lambda: (0, 0)),
            pl.BlockSpec((1, HIDDEN), lambda: (0, 0)),
        ],
        out_specs=pl.BlockSpec((N_NODES, HIDDEN), lambda: (0, 0)),
        out_shape=jax.ShapeDtypeStruct((N_NODES, HIDDEN), jnp.float32),
    )(x, aggs, W1, b1.reshape(1, HIDDEN), W2, b2.reshape(1, HIDDEN),
      gamma.reshape(1, HIDDEN), beta.reshape(1, HIDDEN))


def kernel(x, edge_index, edge_attr, W_e, b_e, W1, b1, W2, b2, gamma, beta):
    src = edge_index[0].astype(jnp.int32)
    dst = edge_index[1].astype(jnp.int32)
    e = _edge_mlp(edge_attr, W_e, b_e)
    aggs = _sc_aggregate(x, src, dst, e)[:, :N_NODES]
    return _node_mlp(x, aggs, W1, b1, W2, b2, gamma, beta)
